# baseline (speedup 1.0000x reference)
import functools

import jax
import jax.numpy as jnp
import numpy as np
from jax import lax
from jax.experimental import pallas as pl
from jax.experimental.pallas import tpu as pltpu
from jax.experimental.pallas import tpu_sc as plsc

N_LORS = 50000
NC, NS, L = 2, 16, 16
LORS_PER_TILE = 3136
N_PAD = NS * LORS_PER_TILE
GROUPS = LORS_PER_TILE // L
NZ_LOC = 64
SLAB_WORDS = NZ_LOC * 128 * 128
TILE_WORDS = SLAB_WORDS // NS
CHUNK = 1024

VOX = 3.125
INV_VOX = 1.0 / VOX
KERNEL_WIDTH = float(np.sqrt(3.0 * 3.0 * np.pi))
NEG_INV_2SIG2 = -1.0 / (2.0 * (KERNEL_WIDTH * 0.5) ** 2)

_COEFS = {"x": (128, 1, 128 * 128), "y": (NZ_LOC * 128, 1, 128),
          "z": (NZ_LOC * 128, NZ_LOC, 1)}
_AXES = ("x", "y", "z")


def _zero_fill(buf, words):
    def body(i, carry):
        buf[pl.ds(i * L, L)] = jnp.zeros((L,), jnp.float32)
        return carry
    lax.fori_loop(0, words // L, body, 0)


def _body(*refs):
    ins = refs[:21]
    outs = refs[21:24]
    (spmem, st0, st1, st2, st3, st4, st5, st6,
     idx_a, val_a, idx_b, val_b, bounce0, bounce1, zero_buf,
     sem_a, sem_b, wb_sem0, wb_sem1, zr_sem) = refs[24:]
    sts = (st0, st1, st2, st3, st4, st5, st6)

    c = lax.axis_index("c")
    s = lax.axis_index("s")
    t0 = s * TILE_WORDS
    base = s * LORS_PER_TILE

    _zero_fill(zero_buf, 8192)

    def stage(ax):
        for j in range(7):
            pltpu.sync_copy(ins[ax * 7 + j].at[pl.ds(base, LORS_PER_TILE)], sts[j])

    stage(0)
    for i in range(TILE_WORDS // 8192):
        pltpu.async_copy(zero_buf, spmem.at[pl.ds(t0 + i * 8192, 8192)], zr_sem)
    for i in range(TILE_WORDS // 8192):
        pltpu.make_async_copy(zero_buf, spmem.at[pl.ds(t0 + i * 8192, 8192)],
                              zr_sem).wait()
    plsc.subcore_barrier()

    zbase = -198.4375 + c.astype(jnp.float32) * 200.0

    for ax, axis_name in enumerate(_AXES):
        ca, cb, cc = _COEFS[axis_name]

        def group_body(g, carry, _ca=ca, _cb=cb, _cc=cc):
            o = g * L
            p1x = st0[pl.ds(o, L)]
            p1y = st1[pl.ds(o, L)]
            p1z = st2[pl.ds(o, L)]
            dx = st3[pl.ds(o, L)] - p1x
            dy = st4[pl.ds(o, L)] - p1y
            dz = st5[pl.ds(o, L)] - p1z
            proj = st6[pl.ds(o, L)]
            dz = jnp.where(jnp.abs(dz) < 1e-6, jnp.float32(1e-6), dz)
            inv_dz = 1.0 / dz
            t0v = (zbase - p1z) * inv_dz
            ts = (VOX * inv_dz)
            fx0 = (p1x + t0v * dx + 200.0) * INV_VOX
            fxs = ts * dx * INV_VOX
            fy0 = (p1y + t0v * dy + 200.0) * INV_VOX
            fys = ts * dy * INV_VOX
            c2 = VOX * VOX * NEG_INV_2SIG2
            for zl in range(64):
                fx = fx0 + fxs * float(zl)
                fy = fy0 + fys * float(zl)
                ixi = fx.astype(jnp.int32)
                iyi = fy.astype(jnp.int32)
                ax = fx - ixi.astype(jnp.float32) - 0.5
                ay = fy - iyi.astype(jnp.float32) - 0.5
                w = jnp.exp((ax * ax + ay * ay) * c2)
                idx_a[pl.ds(zl * L, L)] = ixi * _ca + iyi * _cb + zl * _cc
                val_a[pl.ds(zl * L, L)] = w * proj
            pltpu.sync_copy(val_a, spmem.at[idx_a], add=True)
            return carry
        lax.fori_loop(0, GROUPS, group_body, 0)
        if ax < 2:
            stage(ax + 1)
        plsc.subcore_barrier()

        bounces = (bounce0, bounce1)
        rd_sems = (sem_a, sem_b)
        wb_sems = (wb_sem0, wb_sem1)
        nchunks = TILE_WORDS // 8192

        def rd(i, wait=False):
            chunk = spmem.at[pl.ds(t0 + i * 8192, 8192)]
            if wait:
                pltpu.make_async_copy(chunk, bounces[i % 2], rd_sems[i % 2]).wait()
            else:
                pltpu.async_copy(chunk, bounces[i % 2], rd_sems[i % 2])

        def zr(i, wait=False):
            chunk = spmem.at[pl.ds(t0 + i * 8192, 8192)]
            if wait:
                pltpu.make_async_copy(zero_buf, chunk, zr_sem).wait()
            else:
                pltpu.async_copy(zero_buf, chunk, zr_sem)

        def wb(i, wait=False):
            bp, sm = bounces[i % 2], wb_sems[i % 2]
            if axis_name == "x":
                dst = outs[ax].at[pl.ds((c * NZ_LOC + s * 4) * 16384 + i * 8192, 8192)]
            else:
                dst = outs[ax].at[pl.ds((s * 8 + i) * 16384 + c * 8192, 8192)]
            if wait:
                pltpu.make_async_copy(bp, dst, sm).wait()
            else:
                pltpu.async_copy(bp, dst, sm)

        if axis_name != "z":
            rd(0)
            for i in range(nchunks):
                rd(i, wait=True)
                if ax < 2:
                    zr(i)
                if i + 1 < nchunks:
                    if i >= 1:
                        wb(i - 1, wait=True)
                    rd(i + 1)
                wb(i)
            wb(nchunks - 2, wait=True)
            wb(nchunks - 1, wait=True)
        else:
            rd(0)
            for i in range(nchunks):
                rd(i, wait=True)
                if i + 1 < nchunks:
                    rd(i + 1)

                def wb_body(b, carry, _i=i, _ax=ax, _bp=bounces[i % 2]):
                    descs = []
                    for j in range(16):
                        jj = b * 16 + j
                        r = s * 1024 + _i * 128 + jj
                        descs.append(pltpu.async_copy(
                            _bp.at[pl.ds(jj * NZ_LOC, NZ_LOC)],
                            outs[_ax].at[pl.ds(r * 128 + c * NZ_LOC, NZ_LOC)],
                            wb_sem0))
                    for d in descs:
                        d.wait()
                    return carry
                lax.fori_loop(0, 8, wb_body, 0)
        if ax < 2:
            for i in range(nchunks):
                zr(i, wait=True)
            plsc.subcore_barrier()


def _make_kernel():
    mesh = plsc.VectorSubcoreMesh(core_axis_name="c", subcore_axis_name="s",
                                  num_cores=NC, num_subcores=NS)
    return pl.kernel(
        _body,
        out_type=[jax.ShapeDtypeStruct((128 * 128 * 128,), jnp.float32)] * 3,
        mesh=mesh,
        scratch_types=[
            pltpu.VMEM_SHARED((SLAB_WORDS,), jnp.float32),
        ] + [pltpu.VMEM((LORS_PER_TILE,), jnp.float32)] * 7 + [
            pltpu.VMEM((CHUNK,), jnp.int32),
            pltpu.VMEM((CHUNK,), jnp.float32),
            pltpu.VMEM((CHUNK,), jnp.int32),
            pltpu.VMEM((CHUNK,), jnp.float32),
            pltpu.VMEM((8192,), jnp.float32),
            pltpu.VMEM((8192,), jnp.float32),
            pltpu.VMEM((8192,), jnp.float32),
            pltpu.SemaphoreType.DMA,
            pltpu.SemaphoreType.DMA,
            pltpu.SemaphoreType.DMA,
            pltpu.SemaphoreType.DMA,
            pltpu.SemaphoreType.DMA,
        ],
    )


_ROTATIONS = {"x": (1, 2, 0), "y": (0, 2, 1), "z": (0, 1, 2)}


@jax.jit
def kernel(image, xlors, ylors, zlors, xproj, yproj, zproj):
    del image
    lors = {"x": xlors, "y": ylors, "z": zlors}
    projs = {"x": xproj, "y": yproj, "z": zproj}
    pad = N_PAD - N_LORS
    args = []
    for a in _AXES:
        perm = _ROTATIONS[a]
        lr = lors[a]
        for j in (perm[0], perm[1], perm[2], perm[0] + 3, perm[1] + 3, perm[2] + 3):
            col = lr[:, j]
            args.append(jnp.concatenate([col, jnp.broadcast_to(col[0], (pad,))]))
        args.append(jnp.concatenate([projs[a], jnp.zeros((pad,), jnp.float32)]))
    o0, o1, o2 = _make_kernel()(*args)
    return (o0.reshape(128, 128, 128), o1.reshape(128, 128, 128),
            o2.reshape(128, 128, 128))


# speedup vs baseline: 1.1554x; 1.1554x over previous
import functools

import jax
import jax.numpy as jnp
import numpy as np
from jax import lax
from jax.experimental import pallas as pl
from jax.experimental.pallas import tpu as pltpu
from jax.experimental.pallas import tpu_sc as plsc

N_LORS = 50000
NC, NS, L = 2, 16, 16
LORS_PER_TILE = 3136
N_PAD = NS * LORS_PER_TILE
GROUPS = LORS_PER_TILE // L
NZ_LOC = 64
SLAB_WORDS = NZ_LOC * 128 * 128
TILE_WORDS = SLAB_WORDS // NS
CHUNK = 1024

VOX = 3.125
INV_VOX = 1.0 / VOX
KERNEL_WIDTH = float(np.sqrt(3.0 * 3.0 * np.pi))
NEG_INV_2SIG2 = -1.0 / (2.0 * (KERNEL_WIDTH * 0.5) ** 2)

_COEFS = {"x": (128, 1, 128 * 128), "y": (NZ_LOC * 128, 1, 128),
          "z": (NZ_LOC * 128, NZ_LOC, 1)}
_AXES = ("x", "y", "z")


def _zero_fill(buf, words):
    def body(i, carry):
        buf[pl.ds(i * L, L)] = jnp.zeros((L,), jnp.float32)
        return carry
    lax.fori_loop(0, words // L, body, 0)


def _zero_fill_i32(buf, words):
    def body(i, carry):
        buf[pl.ds(i * L, L)] = jnp.zeros((L,), jnp.int32)
        return carry
    lax.fori_loop(0, words // L, body, 0)


def _body(*refs):
    ins = refs[:21]
    outs = refs[21:24]
    (spmem, st0, st1, st2, st3, st4, st5, st6,
     idx_a, val_a, idx_b, val_b, bounce0, bounce1, zero_buf,
     sem_a, sem_b, wb_sem0, wb_sem1, zr_sem) = refs[24:]
    sts = (st0, st1, st2, st3, st4, st5, st6)

    c = lax.axis_index("c")
    s = lax.axis_index("s")
    t0 = s * TILE_WORDS
    base = s * LORS_PER_TILE

    _zero_fill(zero_buf, 8192)

    def stage(ax):
        for j in range(7):
            pltpu.sync_copy(ins[ax * 7 + j].at[pl.ds(base, LORS_PER_TILE)], sts[j])

    stage(0)
    for i in range(TILE_WORDS // 8192):
        pltpu.async_copy(zero_buf, spmem.at[pl.ds(t0 + i * 8192, 8192)], zr_sem)
    for i in range(TILE_WORDS // 8192):
        pltpu.make_async_copy(zero_buf, spmem.at[pl.ds(t0 + i * 8192, 8192)],
                              zr_sem).wait()
    plsc.subcore_barrier()

    zbase = -198.4375 + c.astype(jnp.float32) * 200.0

    for ax, axis_name in enumerate(_AXES):
        ca, cb, cc = _COEFS[axis_name]

        _zero_fill(val_a, CHUNK)
        _zero_fill(val_b, CHUNK)
        _zero_fill_i32(idx_a, CHUNK)
        _zero_fill_i32(idx_b, CHUNK)
        pltpu.async_copy(val_a, spmem.at[idx_a], sem_a, add=True)
        pltpu.async_copy(val_b, spmem.at[idx_b], sem_b, add=True)

        def run_group(g, ib, vb, sm, _ca=ca, _cb=cb, _cc=cc):
            o = g * L
            p1x = st0[pl.ds(o, L)]
            p1y = st1[pl.ds(o, L)]
            p1z = st2[pl.ds(o, L)]
            dx = st3[pl.ds(o, L)] - p1x
            dy = st4[pl.ds(o, L)] - p1y
            dz = st5[pl.ds(o, L)] - p1z
            proj = st6[pl.ds(o, L)]
            dz = jnp.where(jnp.abs(dz) < 1e-6, jnp.float32(1e-6), dz)
            inv_dz = 1.0 / dz
            t0v = (zbase - p1z) * inv_dz
            ts = (VOX * inv_dz)
            fx0 = (p1x + t0v * dx + 200.0) * INV_VOX
            fxs = ts * dx * INV_VOX
            fy0 = (p1y + t0v * dy + 200.0) * INV_VOX
            fys = ts * dy * INV_VOX
            c2 = VOX * VOX * NEG_INV_2SIG2
            pltpu.make_async_copy(vb, spmem.at[ib], sm).wait()

            def zb_body(zb, carry):
                zb8 = zb * 8
                fxb = fx0 + fxs * zb8.astype(jnp.float32)
                fyb = fy0 + fys * zb8.astype(jnp.float32)
                zoff = zb8 * _cc
                for zz in range(8):
                    fx = fxb + fxs * float(zz)
                    fy = fyb + fys * float(zz)
                    ixi = fx.astype(jnp.int32)
                    iyi = fy.astype(jnp.int32)
                    axq = fx - ixi.astype(jnp.float32) - 0.5
                    ayq = fy - iyi.astype(jnp.float32) - 0.5
                    w = jnp.exp((axq * axq + ayq * ayq) * c2)
                    o2 = zb * 128 + zz * L
                    ib[pl.ds(o2, L)] = ixi * _ca + iyi * _cb + (zoff + zz * _cc)
                    vb[pl.ds(o2, L)] = w * proj
                return carry
            lax.fori_loop(0, 8, zb_body, 0)
            pltpu.async_copy(vb, spmem.at[ib], sm, add=True)

        def pair_body(i, carry):
            run_group(2 * i, idx_a, val_a, sem_a)
            run_group(2 * i + 1, idx_b, val_b, sem_b)
            return carry
        lax.fori_loop(0, GROUPS // 2, pair_body, 0)
        pltpu.make_async_copy(val_a, spmem.at[idx_a], sem_a).wait()
        pltpu.make_async_copy(val_b, spmem.at[idx_b], sem_b).wait()
        if ax < 2:
            stage(ax + 1)
        plsc.subcore_barrier()

        bounces = (bounce0, bounce1)
        rd_sems = (sem_a, sem_b)
        wb_sems = (wb_sem0, wb_sem1)
        nchunks = TILE_WORDS // 8192

        def rd(i, wait=False):
            chunk = spmem.at[pl.ds(t0 + i * 8192, 8192)]
            if wait:
                pltpu.make_async_copy(chunk, bounces[i % 2], rd_sems[i % 2]).wait()
            else:
                pltpu.async_copy(chunk, bounces[i % 2], rd_sems[i % 2])

        def zr(i, wait=False):
            chunk = spmem.at[pl.ds(t0 + i * 8192, 8192)]
            if wait:
                pltpu.make_async_copy(zero_buf, chunk, zr_sem).wait()
            else:
                pltpu.async_copy(zero_buf, chunk, zr_sem)

        def wb(i, wait=False):
            bp, sm = bounces[i % 2], wb_sems[i % 2]
            if axis_name == "x":
                dst = outs[ax].at[pl.ds((c * NZ_LOC + s * 4) * 16384 + i * 8192, 8192)]
            else:
                dst = outs[ax].at[pl.ds((s * 8 + i) * 16384 + c * 8192, 8192)]
            if wait:
                pltpu.make_async_copy(bp, dst, sm).wait()
            else:
                pltpu.async_copy(bp, dst, sm)

        if axis_name != "z":
            rd(0)
            for i in range(nchunks):
                rd(i, wait=True)
                if ax < 2:
                    zr(i)
                if i + 1 < nchunks:
                    if i >= 1:
                        wb(i - 1, wait=True)
                    rd(i + 1)
                wb(i)
            wb(nchunks - 2, wait=True)
            wb(nchunks - 1, wait=True)
        else:
            rd(0)
            for i in range(nchunks):
                rd(i, wait=True)
                if i + 1 < nchunks:
                    rd(i + 1)

                def wb_body(b, carry, _i=i, _ax=ax, _bp=bounces[i % 2]):
                    descs = []
                    for j in range(16):
                        jj = b * 16 + j
                        r = s * 1024 + _i * 128 + jj
                        descs.append(pltpu.async_copy(
                            _bp.at[pl.ds(jj * NZ_LOC, NZ_LOC)],
                            outs[_ax].at[pl.ds(r * 128 + c * NZ_LOC, NZ_LOC)],
                            wb_sem0))
                    for d in descs:
                        d.wait()
                    return carry
                lax.fori_loop(0, 8, wb_body, 0)
        if ax < 2:
            for i in range(nchunks):
                zr(i, wait=True)
            plsc.subcore_barrier()


def _make_kernel():
    mesh = plsc.VectorSubcoreMesh(core_axis_name="c", subcore_axis_name="s",
                                  num_cores=NC, num_subcores=NS)
    return pl.kernel(
        _body,
        out_type=[jax.ShapeDtypeStruct((128 * 128 * 128,), jnp.float32)] * 3,
        mesh=mesh,
        scratch_types=[
            pltpu.VMEM_SHARED((SLAB_WORDS,), jnp.float32),
        ] + [pltpu.VMEM((LORS_PER_TILE,), jnp.float32)] * 7 + [
            pltpu.VMEM((CHUNK,), jnp.int32),
            pltpu.VMEM((CHUNK,), jnp.float32),
            pltpu.VMEM((CHUNK,), jnp.int32),
            pltpu.VMEM((CHUNK,), jnp.float32),
            pltpu.VMEM((8192,), jnp.float32),
            pltpu.VMEM((8192,), jnp.float32),
            pltpu.VMEM((8192,), jnp.float32),
            pltpu.SemaphoreType.DMA,
            pltpu.SemaphoreType.DMA,
            pltpu.SemaphoreType.DMA,
            pltpu.SemaphoreType.DMA,
            pltpu.SemaphoreType.DMA,
        ],
    )


_ROTATIONS = {"x": (1, 2, 0), "y": (0, 2, 1), "z": (0, 1, 2)}


@jax.jit
def kernel(image, xlors, ylors, zlors, xproj, yproj, zproj):
    del image
    lors = {"x": xlors, "y": ylors, "z": zlors}
    projs = {"x": xproj, "y": yproj, "z": zproj}
    pad = N_PAD - N_LORS
    args = []
    for a in _AXES:
        perm = _ROTATIONS[a]
        lr = lors[a]
        for j in (perm[0], perm[1], perm[2], perm[0] + 3, perm[1] + 3, perm[2] + 3):
            col = lr[:, j]
            args.append(jnp.concatenate([col, jnp.broadcast_to(col[0], (pad,))]))
        args.append(jnp.concatenate([projs[a], jnp.zeros((pad,), jnp.float32)]))
    o0, o1, o2 = _make_kernel()(*args)
    return (o0.reshape(128, 128, 128), o1.reshape(128, 128, 128),
            o2.reshape(128, 128, 128))


# speedup vs baseline: 1.4042x; 1.2154x over previous
import functools

import jax
import jax.numpy as jnp
import numpy as np
from jax import lax
from jax.experimental import pallas as pl
from jax.experimental.pallas import tpu as pltpu
from jax.experimental.pallas import tpu_sc as plsc

N_LORS = 50000
NC, NS, L = 2, 16, 16
LORS_PER_TILE = 3136
N_PAD = NS * LORS_PER_TILE
GROUPS = LORS_PER_TILE // L
NZ_LOC = 64
SLAB_WORDS = NZ_LOC * 128 * 128
TILE_WORDS = SLAB_WORDS // NS
CHUNK = 128

VOX = 3.125
INV_VOX = 1.0 / VOX
KERNEL_WIDTH = float(np.sqrt(3.0 * 3.0 * np.pi))
NEG_INV_2SIG2 = -1.0 / (2.0 * (KERNEL_WIDTH * 0.5) ** 2)

_COEFS = {"x": (128, 1, 128 * 128), "y": (NZ_LOC * 128, 1, 128),
          "z": (NZ_LOC * 128, NZ_LOC, 1)}
_AXES = ("x", "y", "z")


def _zero_fill(buf, words):
    def body(i, carry):
        buf[pl.ds(i * L, L)] = jnp.zeros((L,), jnp.float32)
        return carry
    lax.fori_loop(0, words // L, body, 0)


def _body(*refs):
    ins = refs[:21]
    outs = refs[21:24]
    (spmem, st0, st1, st2, st3, st4, st5, st6,
     idx_a, val_a, idx_b, val_b, idx_c, val_c, idx_d, val_d,
     bounce0, bounce1, zero_buf,
     sem_a, sem_b, sem_c, sem_d, wb_sem0, wb_sem1, zr_sem) = refs[24:]
    sts = (st0, st1, st2, st3, st4, st5, st6)

    c = lax.axis_index("c")
    s = lax.axis_index("s")
    t0 = s * TILE_WORDS
    base = s * LORS_PER_TILE

    _zero_fill(zero_buf, 8192)

    def stage(ax):
        for j in range(7):
            pltpu.sync_copy(ins[ax * 7 + j].at[pl.ds(base, LORS_PER_TILE)], sts[j])

    stage(0)
    for i in range(TILE_WORDS // 8192):
        pltpu.async_copy(zero_buf, spmem.at[pl.ds(t0 + i * 8192, 8192)], zr_sem)
    for i in range(TILE_WORDS // 8192):
        pltpu.make_async_copy(zero_buf, spmem.at[pl.ds(t0 + i * 8192, 8192)],
                              zr_sem).wait()
    plsc.subcore_barrier()

    zbase = -198.4375 + c.astype(jnp.float32) * 200.0

    for ax, axis_name in enumerate(_AXES):
        ca, cb, cc = _COEFS[axis_name]

        ring = ((idx_a, val_a, sem_a), (idx_b, val_b, sem_b),
                (idx_c, val_c, sem_c), (idx_d, val_d, sem_d))
        for ibp, vbp, smp in ring:
            _zero_fill(vbp, CHUNK)
            for k in range(CHUNK // L):
                ibp[pl.ds(k * L, L)] = jnp.zeros((L,), jnp.int32)
            pltpu.async_copy(vbp, spmem.at[ibp], smp, add=True)

        def group_body(g, carry, _ca=ca, _cb=cb, _cc=cc):
            o = g * L
            p1x = st0[pl.ds(o, L)]
            p1y = st1[pl.ds(o, L)]
            p1z = st2[pl.ds(o, L)]
            dx = st3[pl.ds(o, L)] - p1x
            dy = st4[pl.ds(o, L)] - p1y
            dz = st5[pl.ds(o, L)] - p1z
            proj = st6[pl.ds(o, L)]
            dz = jnp.where(jnp.abs(dz) < 1e-6, jnp.float32(1e-6), dz)
            inv_dz = 1.0 / dz
            t0v = (zbase - p1z) * inv_dz
            ts = (VOX * inv_dz)
            fx0 = (p1x + t0v * dx + 200.0) * INV_VOX
            fxs = ts * dx * INV_VOX
            fy0 = (p1y + t0v * dy + 200.0) * INV_VOX
            fys = ts * dy * INV_VOX
            c2 = VOX * VOX * NEG_INV_2SIG2
            for zb in range(8):
                ib, vb, sm = ring[zb % 4]
                pltpu.make_async_copy(vb, spmem.at[ib], sm).wait()
                for zz in range(8):
                    zl = zb * 8 + zz
                    fx = fx0 + fxs * float(zl)
                    fy = fy0 + fys * float(zl)
                    ixi = fx.astype(jnp.int32)
                    iyi = fy.astype(jnp.int32)
                    ax = fx - ixi.astype(jnp.float32) - 0.5
                    ay = fy - iyi.astype(jnp.float32) - 0.5
                    w = jnp.exp((ax * ax + ay * ay) * c2)
                    ib[pl.ds(zz * L, L)] = ixi * _ca + iyi * _cb + zl * _cc
                    vb[pl.ds(zz * L, L)] = w * proj
                pltpu.async_copy(vb, spmem.at[ib], sm, add=True)
            return carry
        lax.fori_loop(0, GROUPS, group_body, 0)
        for ibp, vbp, smp in ring:
            pltpu.make_async_copy(vbp, spmem.at[ibp], smp).wait()
        if ax < 2:
            stage(ax + 1)
        plsc.subcore_barrier()

        bounces = (bounce0, bounce1)
        rd_sems = (sem_a, sem_b)
        wb_sems = (wb_sem0, wb_sem1)
        nchunks = TILE_WORDS // 8192

        def rd(i, wait=False):
            chunk = spmem.at[pl.ds(t0 + i * 8192, 8192)]
            if wait:
                pltpu.make_async_copy(chunk, bounces[i % 2], rd_sems[i % 2]).wait()
            else:
                pltpu.async_copy(chunk, bounces[i % 2], rd_sems[i % 2])

        def zr(i, wait=False):
            chunk = spmem.at[pl.ds(t0 + i * 8192, 8192)]
            if wait:
                pltpu.make_async_copy(zero_buf, chunk, zr_sem).wait()
            else:
                pltpu.async_copy(zero_buf, chunk, zr_sem)

        def wb(i, wait=False):
            bp, sm = bounces[i % 2], wb_sems[i % 2]
            if axis_name == "x":
                dst = outs[ax].at[pl.ds((c * NZ_LOC + s * 4) * 16384 + i * 8192, 8192)]
            else:
                dst = outs[ax].at[pl.ds((s * 8 + i) * 16384 + c * 8192, 8192)]
            if wait:
                pltpu.make_async_copy(bp, dst, sm).wait()
            else:
                pltpu.async_copy(bp, dst, sm)

        if axis_name != "z":
            rd(0)
            for i in range(nchunks):
                rd(i, wait=True)
                if ax < 2:
                    zr(i)
                if i + 1 < nchunks:
                    if i >= 1:
                        wb(i - 1, wait=True)
                    rd(i + 1)
                wb(i)
            wb(nchunks - 2, wait=True)
            wb(nchunks - 1, wait=True)
        else:
            rd(0)
            for i in range(nchunks):
                rd(i, wait=True)
                if i + 1 < nchunks:
                    rd(i + 1)

                def wb_body(b, carry, _i=i, _ax=ax, _bp=bounces[i % 2]):
                    descs = []
                    for j in range(16):
                        jj = b * 16 + j
                        r = s * 1024 + _i * 128 + jj
                        descs.append(pltpu.async_copy(
                            _bp.at[pl.ds(jj * NZ_LOC, NZ_LOC)],
                            outs[_ax].at[pl.ds(r * 128 + c * NZ_LOC, NZ_LOC)],
                            wb_sem0))
                    for d in descs:
                        d.wait()
                    return carry
                lax.fori_loop(0, 8, wb_body, 0)
        if ax < 2:
            for i in range(nchunks):
                zr(i, wait=True)
            plsc.subcore_barrier()


def _make_kernel():
    mesh = plsc.VectorSubcoreMesh(core_axis_name="c", subcore_axis_name="s",
                                  num_cores=NC, num_subcores=NS)
    return pl.kernel(
        _body,
        out_type=[jax.ShapeDtypeStruct((128 * 128 * 128,), jnp.float32)] * 3,
        mesh=mesh,
        scratch_types=[
            pltpu.VMEM_SHARED((SLAB_WORDS,), jnp.float32),
        ] + [pltpu.VMEM((LORS_PER_TILE,), jnp.float32)] * 7 + [
            pltpu.VMEM((CHUNK,), jnp.int32),
            pltpu.VMEM((CHUNK,), jnp.float32),
            pltpu.VMEM((CHUNK,), jnp.int32),
            pltpu.VMEM((CHUNK,), jnp.float32),
            pltpu.VMEM((CHUNK,), jnp.int32),
            pltpu.VMEM((CHUNK,), jnp.float32),
            pltpu.VMEM((CHUNK,), jnp.int32),
            pltpu.VMEM((CHUNK,), jnp.float32),
            pltpu.VMEM((8192,), jnp.float32),
            pltpu.VMEM((8192,), jnp.float32),
            pltpu.VMEM((8192,), jnp.float32),
            pltpu.SemaphoreType.DMA,
            pltpu.SemaphoreType.DMA,
            pltpu.SemaphoreType.DMA,
            pltpu.SemaphoreType.DMA,
            pltpu.SemaphoreType.DMA,
            pltpu.SemaphoreType.DMA,
            pltpu.SemaphoreType.DMA,
        ],
    )


_ROTATIONS = {"x": (1, 2, 0), "y": (0, 2, 1), "z": (0, 1, 2)}


@jax.jit
def kernel(image, xlors, ylors, zlors, xproj, yproj, zproj):
    del image
    lors = {"x": xlors, "y": ylors, "z": zlors}
    projs = {"x": xproj, "y": yproj, "z": zproj}
    pad = N_PAD - N_LORS
    args = []
    for a in _AXES:
        perm = _ROTATIONS[a]
        lr = lors[a]
        for j in (perm[0], perm[1], perm[2], perm[0] + 3, perm[1] + 3, perm[2] + 3):
            col = lr[:, j]
            args.append(jnp.concatenate([col, jnp.broadcast_to(col[0], (pad,))]))
        args.append(jnp.concatenate([projs[a], jnp.zeros((pad,), jnp.float32)]))
    o0, o1, o2 = _make_kernel()(*args)
    return (o0.reshape(128, 128, 128), o1.reshape(128, 128, 128),
            o2.reshape(128, 128, 128))
